# baseline (speedup 1.0000x reference)
import functools

import jax
import jax.numpy as jnp
from jax import lax
from jax.experimental import pallas as pl
from jax.experimental.pallas import tpu as pltpu
from jax.experimental.pallas import tpu_sc as plsc

_B = 16384
_P = 10
_D = 64
_NW = 32

_POI_PER_W = _B // _NW
_CBG_TOT = 6 * _B
_CBG_PER_W = _CBG_TOT // _NW
_CHUNK = 1024

_BLK = 2048
_GRID = _B // _BLK


def _gather_body(poi_emb, cbg_emb, poi_ids, cbg_ids, poi_out, cbg_out,
                 pidx_v, prow_v, cidx_v, crow_v, sem):
    wid = lax.axis_index("s") * 2 + lax.axis_index("c")
    pbase = wid * _POI_PER_W
    pltpu.sync_copy(poi_ids.at[pl.ds(pbase, _POI_PER_W)], pidx_v)
    pltpu.async_copy(poi_emb.at[pidx_v], prow_v, sem).wait()
    pltpu.sync_copy(prow_v, poi_out.at[pl.ds(pbase, _POI_PER_W)])
    cbase = wid * _CBG_PER_W
    for i in range(_CBG_PER_W // _CHUNK):
        off = cbase + i * _CHUNK
        pltpu.sync_copy(cbg_ids.at[pl.ds(off, _CHUNK)], cidx_v)
        pltpu.async_copy(cbg_emb.at[cidx_v], crow_v, sem).wait()
        pltpu.sync_copy(crow_v, cbg_out.at[pl.ds(off, _CHUNK)])


@functools.cache
def _gather_sc():
    return pl.kernel(
        _gather_body,
        mesh=plsc.VectorSubcoreMesh(core_axis_name="c", subcore_axis_name="s"),
        out_type=[
            jax.ShapeDtypeStruct((_B, _D), jnp.float32),
            jax.ShapeDtypeStruct((_CBG_TOT, _D), jnp.float32),
        ],
        scratch_types=[
            pltpu.VMEM((_POI_PER_W,), jnp.int32),
            pltpu.VMEM((_POI_PER_W, _D), jnp.float32),
            pltpu.VMEM((_CHUNK,), jnp.int32),
            pltpu.VMEM((_CHUNK, _D), jnp.float32),
            pltpu.SemaphoreType.DMA,
        ],
        compiler_params=pltpu.CompilerParams(use_tc_tiling_on_sc=False),
    )


def _log_sigmoid(t):
    return jnp.minimum(t, 0.0) - jnp.log(1.0 + jnp.exp(-jnp.abs(t)))


def _loss_body(x_ref, cate_emb_ref, perc_emb_ref, poi_ref, cbg_ref, out_ref):
    x = x_ref[...]
    cate_f = x[:, 0:1]
    cats = lax.broadcasted_iota(jnp.int32, (1, 4), 1).astype(jnp.float32)
    oh = (cate_f == cats).astype(jnp.float32)
    cate_e = jnp.dot(oh, cate_emb_ref[...],
                     preferred_element_type=jnp.float32)
    poi_e = poi_ref[...]

    total = jnp.zeros((), jnp.float32)

    for j in range(6):
        c = cbg_ref[j]
        d1 = jnp.sum(cate_e * c, axis=1)
        d2 = jnp.sum(poi_e * c, axis=1)
        if j == 0:
            total += jnp.sum(-_log_sigmoid(d1) - _log_sigmoid(d2))
        else:
            total += 0.2 * jnp.sum(-_log_sigmoid(-d1) - _log_sigmoid(-d2))

    percs = (1.0 / (2.0 * _P)
             + lax.broadcasted_iota(jnp.int32, (1, _P), 1).astype(jnp.float32) / _P)
    for i in range(5):
        for sign, col in ((1.0, 3 + i), (-1.0, 9 + 2 * i)):
            fv = x[:, col:col + 1]
            logits = -jnp.abs(_P * (fv - percs))
            z = logits - jnp.max(logits, axis=1, keepdims=True)
            e = jnp.exp(z)
            m = e / jnp.sum(e, axis=1, keepdims=True)
            blockw = perc_emb_ref[i * _P:(i + 1) * _P, :]
            emb = jnp.dot(m, blockw, preferred_element_type=jnp.float32)
            d1 = jnp.sum(cate_e * emb, axis=1)
            d2 = jnp.sum(poi_e * emb, axis=1)
            total += jnp.sum(-_log_sigmoid(sign * d1) - _log_sigmoid(sign * d2))

    @pl.when(pl.program_id(0) == 0)
    def _init():
        out_ref[0, 0] = 0.0

    out_ref[0, 0] += total


_loss_tc = pl.pallas_call(
    _loss_body,
    grid=(_GRID,),
    in_specs=[
        pl.BlockSpec((_BLK, 18), lambda i: (i, 0)),
        pl.BlockSpec((4, _D), lambda i: (0, 0)),
        pl.BlockSpec((5 * _P, _D), lambda i: (0, 0)),
        pl.BlockSpec((_BLK, _D), lambda i: (i, 0)),
        pl.BlockSpec((6, _BLK, _D), lambda i: (0, i, 0)),
    ],
    out_specs=pl.BlockSpec(
        (1, 1), lambda i: (0, 0), memory_space=pltpu.SMEM),
    out_shape=jax.ShapeDtypeStruct((1, 1), jnp.float32),
    compiler_params=pltpu.CompilerParams(
        dimension_semantics=("arbitrary",)),
)


def kernel(inputs, cate_emb, poi_emb, cbg_emb, perc_emb):
    poi_ids = inputs[:, 1].astype(jnp.int32)
    cbg_cols = [2, 8, 10, 12, 14, 16]
    cbg_ids = jnp.concatenate(
        [inputs[:, c] for c in cbg_cols]).astype(jnp.int32)
    poi_rows, cbg_rows = _gather_sc()(poi_emb, cbg_emb, poi_ids, cbg_ids)
    cbg_rows = cbg_rows.reshape(6, _B, _D)
    out = _loss_tc(inputs, cate_emb, perc_emb, poi_rows, cbg_rows)
    return out[0, 0]


# speedup vs baseline: 1.1787x; 1.1787x over previous
import functools

import jax
import jax.numpy as jnp
from jax import lax
from jax.experimental import pallas as pl
from jax.experimental.pallas import tpu as pltpu
from jax.experimental.pallas import tpu_sc as plsc

_B = 16384
_P = 10
_D = 64
_NW = 32

_POI_PER_W = _B // _NW
_CBG_TOT = 6 * _B
_CBG_PER_W = _CBG_TOT // _NW
_CHUNK = 1024

_BLK = 2048
_GRID = _B // _BLK


def _gather_body(poi_emb, cbg_emb, poi_ids, cbg_ids, poi_out, cbg_out,
                 pidx_v, prow_v, cidx_v, crow_v, sem):
    wid = lax.axis_index("s") * 2 + lax.axis_index("c")
    pbase = wid * _POI_PER_W
    pltpu.sync_copy(poi_ids.at[pl.ds(pbase, _POI_PER_W)], pidx_v)
    pltpu.async_copy(poi_emb.at[pidx_v], prow_v, sem).wait()
    pltpu.sync_copy(prow_v, poi_out.at[pl.ds(pbase, _POI_PER_W)])
    cbase = wid * _CBG_PER_W
    for i in range(_CBG_PER_W // _CHUNK):
        off = cbase + i * _CHUNK
        pltpu.sync_copy(cbg_ids.at[pl.ds(off, _CHUNK)], cidx_v)
        pltpu.async_copy(cbg_emb.at[cidx_v], crow_v, sem).wait()
        pltpu.sync_copy(crow_v, cbg_out.at[pl.ds(off, _CHUNK)])


@functools.cache
def _gather_sc():
    return pl.kernel(
        _gather_body,
        mesh=plsc.VectorSubcoreMesh(core_axis_name="c", subcore_axis_name="s"),
        out_type=[
            jax.ShapeDtypeStruct((_B, _D), jnp.float32),
            jax.ShapeDtypeStruct((_CBG_TOT, _D), jnp.float32),
        ],
        scratch_types=[
            pltpu.VMEM((_POI_PER_W,), jnp.int32),
            pltpu.VMEM((_POI_PER_W, _D), jnp.float32),
            pltpu.VMEM((_CHUNK,), jnp.int32),
            pltpu.VMEM((_CHUNK, _D), jnp.float32),
            pltpu.SemaphoreType.DMA,
        ],
        compiler_params=pltpu.CompilerParams(use_tc_tiling_on_sc=False),
    )


def _log_sigmoid(t):
    return jnp.minimum(t, 0.0) - jnp.log(1.0 + jnp.exp(-jnp.abs(t)))


def _mm(a, b):
    return jax.lax.dot_general(a, b, (((1,), (0,)), ((), ())),
                               preferred_element_type=jnp.float32)


def _mm_t(a, b):
    return jax.lax.dot_general(a, b, (((1,), (1,)), ((), ())),
                               preferred_element_type=jnp.float32)


def _loss_body(x_ref, cate_emb_ref, perc_emb_ref, poi_ref, cbg_ref, out_ref):
    f32 = jnp.float32
    x = x_ref[...]

    r18 = lax.broadcasted_iota(jnp.int32, (18, 5), 0)
    c18 = lax.broadcasted_iota(jnp.int32, (18, 5), 1)
    e_obs = (r18 == 3 + c18).astype(f32)
    e_alt = (r18 == 9 + 2 * c18).astype(f32)
    r5 = lax.broadcasted_iota(jnp.int32, (5, 5 * _P), 0)
    c5 = lax.broadcasted_iota(jnp.int32, (5, 5 * _P), 1)
    rep = (c5 // _P == r5).astype(f32)
    s50 =(lax.broadcasted_iota(jnp.int32, (5 * _P, 5), 0) // _P
           == lax.broadcasted_iota(jnp.int32, (5 * _P, 5), 1)).astype(f32)
    percs = (1.0 / (2.0 * _P)
             + (lax.broadcasted_iota(jnp.int32, (1, 5 * _P), 1) % _P
                ).astype(f32) / _P)
    ones_d = jnp.ones((_D, 1), f32)

    cate_col = x[:, 0:1]
    oh = (cate_col == lax.broadcasted_iota(jnp.int32, (1, 4), 1).astype(f32)
          ).astype(f32)
    cate_e = _mm(oh, cate_emb_ref[...])
    poi_e = poi_ref[...]

    def softmax50(fv5):
        lg = -jnp.abs(_P * (_mm(fv5, rep) - percs))
        e = jnp.exp(lg)
        den = _mm(e, s50)
        return e * _mm(1.0 / den, rep)

    m_obs = softmax50(_mm(x, e_obs))
    m_alt = softmax50(_mm(x, e_alt))

    g_cate = _mm_t(perc_emb_ref[...], cate_emb_ref[...])
    gc_sel = _mm_t(oh, g_cate)
    gp = _mm_t(poi_e, perc_emb_ref[...])
    t_obs_c = _mm(m_obs * gc_sel, s50)
    t_obs_p = _mm(m_obs * gp, s50)
    t_alt_c = _mm(m_alt * gc_sel, s50)
    t_alt_p = _mm(m_alt * gp, s50)

    dots = []
    for j in range(6):
        c = cbg_ref[j]
        sgn = 1.0 if j == 0 else -1.0
        dots.append(_mm(cate_e * c, ones_d) * sgn)
        dots.append(_mm(poi_e * c, ones_d) * sgn)
    packed = jnp.concatenate(
        dots + [t_obs_c, t_obs_p, -t_alt_c, -t_alt_p], axis=1)

    col = lax.broadcasted_iota(jnp.int32, (1, 32), 1)
    w = jnp.where((col >= 2) & (col < 12), 0.2, 1.0)
    total = -jnp.sum(w * _log_sigmoid(packed))

    @pl.when(pl.program_id(0) == 0)
    def _init():
        out_ref[0, 0] = 0.0

    out_ref[0, 0] += total


_loss_tc = pl.pallas_call(
    _loss_body,
    grid=(_GRID,),
    in_specs=[
        pl.BlockSpec((_BLK, 18), lambda i: (i, 0)),
        pl.BlockSpec((4, _D), lambda i: (0, 0)),
        pl.BlockSpec((5 * _P, _D), lambda i: (0, 0)),
        pl.BlockSpec((_BLK, _D), lambda i: (i, 0)),
        pl.BlockSpec((6, _BLK, _D), lambda i: (0, i, 0)),
    ],
    out_specs=pl.BlockSpec(
        (1, 1), lambda i: (0, 0), memory_space=pltpu.SMEM),
    out_shape=jax.ShapeDtypeStruct((1, 1), jnp.float32),
    compiler_params=pltpu.CompilerParams(
        dimension_semantics=("arbitrary",)),
)


def kernel(inputs, cate_emb, poi_emb, cbg_emb, perc_emb):
    poi_ids = inputs[:, 1].astype(jnp.int32)
    cbg_cols = [2, 8, 10, 12, 14, 16]
    cbg_ids = jnp.concatenate(
        [inputs[:, c] for c in cbg_cols]).astype(jnp.int32)
    poi_rows, cbg_rows = _gather_sc()(poi_emb, cbg_emb, poi_ids, cbg_ids)
    cbg_rows = cbg_rows.reshape(6, _B, _D)
    out = _loss_tc(inputs, cate_emb, perc_emb, poi_rows, cbg_rows)
    return out[0, 0]


# speedup vs baseline: 1.6228x; 1.3767x over previous
import functools

import jax
import jax.numpy as jnp
from jax import lax
from jax.experimental import pallas as pl
from jax.experimental.pallas import tpu as pltpu
from jax.experimental.pallas import tpu_sc as plsc

_B = 16384
_P = 10
_D = 64
_NW = 32

_POI_PER_W = _B // _NW
_CBG_TOT = 6 * _B
_CBG_PER_W = _CBG_TOT // _NW
_CHUNK = 1024

_BLK = 2048
_GRID = _B // _BLK


_W = 512
_K = 16


def _gather_block(table, ids_hbm, out_hbm, base, idv, ids_s, rows, sems):
    pltpu.sync_copy(ids_hbm.at[pl.ds(base, _W)], ids_s)
    ngroups = _W // _K
    del idv

    def issue(g):
        idvec = ids_s[pl.ds(g * _K, _K)]
        for k in range(_K):
            pltpu.async_copy(table.at[idvec[k]], rows.at[g * _K + k],
                             sems.at[g % 2])

    def drain(g):
        idvec = ids_s[pl.ds(g * _K, _K)]
        for k in range(_K):
            pltpu.make_async_copy(table.at[idvec[k]], rows.at[g * _K + k],
                                  sems.at[g % 2]).wait()

    issue(0)

    def body(g, carry):
        issue(g + 1)
        drain(g)
        return carry

    lax.fori_loop(0, ngroups - 1, body, 0)
    drain(ngroups - 1)
    pltpu.sync_copy(rows, out_hbm.at[pl.ds(base, _W)])


def _gather_body(poi_emb, cbg_emb, poi_ids, cbg_ids, poi_out, cbg_out,
                 idv, ids_s, rows, sems):
    wid = lax.axis_index("s") * 2 + lax.axis_index("c")
    _gather_block(poi_emb, poi_ids, poi_out, wid * _W, idv, ids_s, rows, sems)
    for j in range(6):
        _gather_block(cbg_emb, cbg_ids, cbg_out, j * _B + wid * _W,
                      idv, ids_s, rows, sems)


@functools.cache
def _gather_sc():
    return pl.kernel(
        _gather_body,
        mesh=plsc.VectorSubcoreMesh(core_axis_name="c", subcore_axis_name="s"),
        out_type=[
            jax.ShapeDtypeStruct((_B, _D), jnp.float32),
            jax.ShapeDtypeStruct((_CBG_TOT, _D), jnp.float32),
        ],
        scratch_types=[
            pltpu.VMEM((_W,), jnp.int32),
            pltpu.VMEM((_W,), jnp.int32),
            pltpu.VMEM((_W, _D), jnp.float32),
            pltpu.SemaphoreType.DMA((2,)),
        ],
        compiler_params=pltpu.CompilerParams(use_tc_tiling_on_sc=True),
    )


def _log_sigmoid(t):
    return jnp.minimum(t, 0.0) - jnp.log(1.0 + jnp.exp(-jnp.abs(t)))


def _mm(a, b):
    return jax.lax.dot_general(a, b, (((1,), (0,)), ((), ())),
                               preferred_element_type=jnp.float32)


def _mm_t(a, b):
    return jax.lax.dot_general(a, b, (((1,), (1,)), ((), ())),
                               preferred_element_type=jnp.float32)


def _loss_body(x_ref, cate_emb_ref, perc_emb_ref, poi_ref, cbg_ref, out_ref):
    f32 = jnp.float32
    x = x_ref[...]

    r18 = lax.broadcasted_iota(jnp.int32, (18, 5), 0)
    c18 = lax.broadcasted_iota(jnp.int32, (18, 5), 1)
    e_obs = (r18 == 3 + c18).astype(f32)
    e_alt = (r18 == 9 + 2 * c18).astype(f32)
    r5 = lax.broadcasted_iota(jnp.int32, (5, 5 * _P), 0)
    c5 = lax.broadcasted_iota(jnp.int32, (5, 5 * _P), 1)
    rep = (c5 // _P == r5).astype(f32)
    s50 =(lax.broadcasted_iota(jnp.int32, (5 * _P, 5), 0) // _P
           == lax.broadcasted_iota(jnp.int32, (5 * _P, 5), 1)).astype(f32)
    percs = (1.0 / (2.0 * _P)
             + (lax.broadcasted_iota(jnp.int32, (1, 5 * _P), 1) % _P
                ).astype(f32) / _P)
    ones_d = jnp.ones((_D, 1), f32)

    cate_col = x[:, 0:1]
    oh = (cate_col == lax.broadcasted_iota(jnp.int32, (1, 4), 1).astype(f32)
          ).astype(f32)
    cate_e = _mm(oh, cate_emb_ref[...])
    poi_e = poi_ref[...]

    def softmax50(fv5):
        lg = -jnp.abs(_P * (_mm(fv5, rep) - percs))
        e = jnp.exp(lg)
        den = _mm(e, s50)
        return e * _mm(1.0 / den, rep)

    m_obs = softmax50(_mm(x, e_obs))
    m_alt = softmax50(_mm(x, e_alt))

    g_cate = _mm_t(perc_emb_ref[...], cate_emb_ref[...])
    gc_sel = _mm_t(oh, g_cate)
    gp = _mm_t(poi_e, perc_emb_ref[...])
    t_obs_c = _mm(m_obs * gc_sel, s50)
    t_obs_p = _mm(m_obs * gp, s50)
    t_alt_c = _mm(m_alt * gc_sel, s50)
    t_alt_p = _mm(m_alt * gp, s50)

    dots = []
    for j in range(6):
        c = cbg_ref[j]
        sgn = 1.0 if j == 0 else -1.0
        dots.append(_mm(cate_e * c, ones_d) * sgn)
        dots.append(_mm(poi_e * c, ones_d) * sgn)
    packed = jnp.concatenate(
        dots + [t_obs_c, t_obs_p, -t_alt_c, -t_alt_p], axis=1)

    col = lax.broadcasted_iota(jnp.int32, (1, 32), 1)
    w = jnp.where((col >= 2) & (col < 12), 0.2, 1.0)
    total = -jnp.sum(w * _log_sigmoid(packed))

    @pl.when(pl.program_id(0) == 0)
    def _init():
        out_ref[0, 0] = 0.0

    out_ref[0, 0] += total


_loss_tc = pl.pallas_call(
    _loss_body,
    grid=(_GRID,),
    in_specs=[
        pl.BlockSpec((_BLK, 18), lambda i: (i, 0)),
        pl.BlockSpec((4, _D), lambda i: (0, 0)),
        pl.BlockSpec((5 * _P, _D), lambda i: (0, 0)),
        pl.BlockSpec((_BLK, _D), lambda i: (i, 0)),
        pl.BlockSpec((6, _BLK, _D), lambda i: (0, i, 0)),
    ],
    out_specs=pl.BlockSpec(
        (1, 1), lambda i: (0, 0), memory_space=pltpu.SMEM),
    out_shape=jax.ShapeDtypeStruct((1, 1), jnp.float32),
    compiler_params=pltpu.CompilerParams(
        dimension_semantics=("arbitrary",)),
)


def kernel(inputs, cate_emb, poi_emb, cbg_emb, perc_emb):
    poi_ids = inputs[:, 1].astype(jnp.int32)
    cbg_cols = [2, 8, 10, 12, 14, 16]
    cbg_ids = jnp.concatenate(
        [inputs[:, c] for c in cbg_cols]).astype(jnp.int32)
    poi_rows, cbg_rows = _gather_sc()(poi_emb, cbg_emb, poi_ids, cbg_ids)
    cbg_rows = cbg_rows.reshape(6, _B, _D)
    out = _loss_tc(inputs, cate_emb, perc_emb, poi_rows, cbg_rows)
    return out[0, 0]


# speedup vs baseline: 1.6264x; 1.0022x over previous
import functools

import jax
import jax.numpy as jnp
from jax import lax
from jax.experimental import pallas as pl
from jax.experimental.pallas import tpu as pltpu
from jax.experimental.pallas import tpu_sc as plsc

_B = 16384
_P = 10
_D = 64
_NW = 32

_POI_PER_W = _B // _NW
_CBG_TOT = 6 * _B
_CBG_PER_W = _CBG_TOT // _NW
_CHUNK = 1024

_BLK = 2048
_GRID = _B // _BLK


_W = 512
_K = 16


def _gather_block(table, ids_hbm, out_hbm, base, idv, ids_s, rows, sems):
    pltpu.sync_copy(ids_hbm.at[pl.ds(base, _W)], ids_s)
    ngroups = _W // _K
    del idv

    def issue(g):
        idvec = ids_s[pl.ds(g * _K, _K)]
        for k in range(_K):
            pltpu.async_copy(table.at[idvec[k]], rows.at[g * _K + k],
                             sems.at[g % 2])

    def drain(g):
        pltpu.make_async_copy(table.at[pl.ds(0, _K)],
                              rows.at[pl.ds(g * _K, _K)],
                              sems.at[g % 2]).wait()

    issue(0)

    def body(g, carry):
        issue(g + 1)
        drain(g)
        return carry

    lax.fori_loop(0, ngroups - 1, body, 0)
    drain(ngroups - 1)
    pltpu.sync_copy(rows, out_hbm.at[pl.ds(base, _W)])


def _gather_body(poi_emb, cbg_emb, poi_ids, cbg_ids, poi_out, cbg_out,
                 idv, ids_s, rows, sems):
    wid = lax.axis_index("s") * 2 + lax.axis_index("c")
    _gather_block(poi_emb, poi_ids, poi_out, wid * _W, idv, ids_s, rows, sems)
    for j in range(6):
        _gather_block(cbg_emb, cbg_ids, cbg_out, j * _B + wid * _W,
                      idv, ids_s, rows, sems)


@functools.cache
def _gather_sc():
    return pl.kernel(
        _gather_body,
        mesh=plsc.VectorSubcoreMesh(core_axis_name="c", subcore_axis_name="s"),
        out_type=[
            jax.ShapeDtypeStruct((_B, _D), jnp.float32),
            jax.ShapeDtypeStruct((_CBG_TOT, _D), jnp.float32),
        ],
        scratch_types=[
            pltpu.VMEM((_W,), jnp.int32),
            pltpu.VMEM((_W,), jnp.int32),
            pltpu.VMEM((_W, _D), jnp.float32),
            pltpu.SemaphoreType.DMA((2,)),
        ],
        compiler_params=pltpu.CompilerParams(use_tc_tiling_on_sc=True),
    )


def _log_sigmoid(t):
    return jnp.minimum(t, 0.0) - jnp.log(1.0 + jnp.exp(-jnp.abs(t)))


def _mm(a, b):
    return jax.lax.dot_general(a, b, (((1,), (0,)), ((), ())),
                               preferred_element_type=jnp.float32)


def _mm_t(a, b):
    return jax.lax.dot_general(a, b, (((1,), (1,)), ((), ())),
                               preferred_element_type=jnp.float32)


def _loss_body(x_ref, cate_emb_ref, perc_emb_ref, poi_ref, cbg_ref, out_ref):
    f32 = jnp.float32
    x = x_ref[...]

    r18 = lax.broadcasted_iota(jnp.int32, (18, 5), 0)
    c18 = lax.broadcasted_iota(jnp.int32, (18, 5), 1)
    e_obs = (r18 == 3 + c18).astype(f32)
    e_alt = (r18 == 9 + 2 * c18).astype(f32)
    r5 = lax.broadcasted_iota(jnp.int32, (5, 5 * _P), 0)
    c5 = lax.broadcasted_iota(jnp.int32, (5, 5 * _P), 1)
    rep = (c5 // _P == r5).astype(f32)
    s50 =(lax.broadcasted_iota(jnp.int32, (5 * _P, 5), 0) // _P
           == lax.broadcasted_iota(jnp.int32, (5 * _P, 5), 1)).astype(f32)
    percs = (1.0 / (2.0 * _P)
             + (lax.broadcasted_iota(jnp.int32, (1, 5 * _P), 1) % _P
                ).astype(f32) / _P)
    ones_d = jnp.ones((_D, 1), f32)

    cate_col = x[:, 0:1]
    oh = (cate_col == lax.broadcasted_iota(jnp.int32, (1, 4), 1).astype(f32)
          ).astype(f32)
    cate_e = _mm(oh, cate_emb_ref[...])
    poi_e = poi_ref[...]

    def softmax50(fv5):
        lg = -jnp.abs(_P * (_mm(fv5, rep) - percs))
        e = jnp.exp(lg)
        den = _mm(e, s50)
        return e * _mm(1.0 / den, rep)

    m_obs = softmax50(_mm(x, e_obs))
    m_alt = softmax50(_mm(x, e_alt))

    g_cate = _mm_t(perc_emb_ref[...], cate_emb_ref[...])
    gc_sel = _mm_t(oh, g_cate)
    gp = _mm_t(poi_e, perc_emb_ref[...])
    t_obs_c = _mm(m_obs * gc_sel, s50)
    t_obs_p = _mm(m_obs * gp, s50)
    t_alt_c = _mm(m_alt * gc_sel, s50)
    t_alt_p = _mm(m_alt * gp, s50)

    dots = []
    for j in range(6):
        c = cbg_ref[j]
        sgn = 1.0 if j == 0 else -1.0
        dots.append(_mm(cate_e * c, ones_d) * sgn)
        dots.append(_mm(poi_e * c, ones_d) * sgn)
    packed = jnp.concatenate(
        dots + [t_obs_c, t_obs_p, -t_alt_c, -t_alt_p], axis=1)

    col = lax.broadcasted_iota(jnp.int32, (1, 32), 1)
    w = jnp.where((col >= 2) & (col < 12), 0.2, 1.0)
    total = -jnp.sum(w * _log_sigmoid(packed))

    @pl.when(pl.program_id(0) == 0)
    def _init():
        out_ref[0, 0] = 0.0

    out_ref[0, 0] += total


_loss_tc = pl.pallas_call(
    _loss_body,
    grid=(_GRID,),
    in_specs=[
        pl.BlockSpec((_BLK, 18), lambda i: (i, 0)),
        pl.BlockSpec((4, _D), lambda i: (0, 0)),
        pl.BlockSpec((5 * _P, _D), lambda i: (0, 0)),
        pl.BlockSpec((_BLK, _D), lambda i: (i, 0)),
        pl.BlockSpec((6, _BLK, _D), lambda i: (0, i, 0)),
    ],
    out_specs=pl.BlockSpec(
        (1, 1), lambda i: (0, 0), memory_space=pltpu.SMEM),
    out_shape=jax.ShapeDtypeStruct((1, 1), jnp.float32),
    compiler_params=pltpu.CompilerParams(
        dimension_semantics=("arbitrary",)),
)


def kernel(inputs, cate_emb, poi_emb, cbg_emb, perc_emb):
    poi_ids = inputs[:, 1].astype(jnp.int32)
    cbg_cols = [2, 8, 10, 12, 14, 16]
    cbg_ids = jnp.concatenate(
        [inputs[:, c] for c in cbg_cols]).astype(jnp.int32)
    poi_rows, cbg_rows = _gather_sc()(poi_emb, cbg_emb, poi_ids, cbg_ids)
    cbg_rows = cbg_rows.reshape(6, _B, _D)
    out = _loss_tc(inputs, cate_emb, perc_emb, poi_rows, cbg_rows)
    return out[0, 0]


# speedup vs baseline: 1.6939x; 1.0415x over previous
import functools

import jax
import jax.numpy as jnp
from jax import lax
from jax.experimental import pallas as pl
from jax.experimental.pallas import tpu as pltpu
from jax.experimental.pallas import tpu_sc as plsc

_B = 16384
_P = 10
_D = 64
_NW = 32

_POI_PER_W = _B // _NW
_CBG_TOT = 6 * _B
_CBG_PER_W = _CBG_TOT // _NW
_CHUNK = 1024

_BLK = 2048
_GRID = _B // _BLK


_W = 512
_PC = 4096
_PR = _PC // 2

_CBG_GRID = (1000000 + _PC - 1) // _PC
_POI_GRID = (100000 + _PC - 1) // _PC


def _pair_body(tin_ref, out_ref):
    x = tin_ref[...]
    xt = jnp.transpose(x, (1, 0))
    out_ref[...] = jnp.concatenate([xt[:_PR], xt[_PR:]], axis=1)


def _make_pair(grid):
    return pl.pallas_call(
        _pair_body,
        grid=(grid,),
        in_specs=[pl.BlockSpec((_D, _PC), lambda i: (0, i))],
        out_specs=pl.BlockSpec((_PR, 2 * _D), lambda i: (i, 0)),
        out_shape=jax.ShapeDtypeStruct((grid * _PR, 2 * _D), jnp.float32),
        compiler_params=pltpu.CompilerParams(
            dimension_semantics=("arbitrary",)),
    )


def _sg_block(table, idx_hbm, out_hbm, base, idxv, rows, sem):
    pltpu.sync_copy(idx_hbm.at[pl.ds(base, _W)], idxv)
    pltpu.async_copy(table.at[idxv], rows, sem).wait()
    pltpu.sync_copy(rows, out_hbm.at[pl.ds(base, _W)])


def _gather_body(poi_pair, cbg_pair, poi_idx, cbg_idx, poi_out, cbg_out,
                 idxv, rows, sem):
    wid = lax.axis_index("s") * 2 + lax.axis_index("c")
    _sg_block(poi_pair, poi_idx, poi_out, wid * _W, idxv, rows, sem)
    for j in range(6):
        _sg_block(cbg_pair, cbg_idx, cbg_out, j * _B + wid * _W,
                  idxv, rows, sem)


@functools.cache
def _gather_sc():
    return pl.kernel(
        _gather_body,
        mesh=plsc.VectorSubcoreMesh(core_axis_name="c", subcore_axis_name="s"),
        out_type=[
            jax.ShapeDtypeStruct((_B, 2 * _D), jnp.float32),
            jax.ShapeDtypeStruct((_CBG_TOT, 2 * _D), jnp.float32),
        ],
        scratch_types=[
            pltpu.VMEM((_W,), jnp.int32),
            pltpu.VMEM((_W, 2 * _D), jnp.float32),
            pltpu.SemaphoreType.DMA,
        ],
        compiler_params=pltpu.CompilerParams(use_tc_tiling_on_sc=True),
    )


def _log_sigmoid(t):
    return jnp.minimum(t, 0.0) - jnp.log(1.0 + jnp.exp(-jnp.abs(t)))


def _mm(a, b):
    return jax.lax.dot_general(a, b, (((1,), (0,)), ((), ())),
                               preferred_element_type=jnp.float32)


def _mm_t(a, b):
    return jax.lax.dot_general(a, b, (((1,), (1,)), ((), ())),
                               preferred_element_type=jnp.float32)


def _loss_body(x_ref, cate_emb_ref, perc_emb_ref, poi_ref, cbg_ref, par_ref,
               out_ref):
    f32 = jnp.float32
    x = x_ref[...]
    par = par_ref[...]

    def half(xfull, p):
        return jnp.where(p > 0.5, xfull[:, _D:2 * _D], xfull[:, 0:_D])

    r18 = lax.broadcasted_iota(jnp.int32, (18, 5), 0)
    c18 = lax.broadcasted_iota(jnp.int32, (18, 5), 1)
    e_obs = (r18 == 3 + c18).astype(f32)
    e_alt = (r18 == 9 + 2 * c18).astype(f32)
    r5 = lax.broadcasted_iota(jnp.int32, (5, 5 * _P), 0)
    c5 = lax.broadcasted_iota(jnp.int32, (5, 5 * _P), 1)
    rep = (c5 // _P == r5).astype(f32)
    s50 =(lax.broadcasted_iota(jnp.int32, (5 * _P, 5), 0) // _P
           == lax.broadcasted_iota(jnp.int32, (5 * _P, 5), 1)).astype(f32)
    percs = (1.0 / (2.0 * _P)
             + (lax.broadcasted_iota(jnp.int32, (1, 5 * _P), 1) % _P
                ).astype(f32) / _P)
    ones_d = jnp.ones((_D, 1), f32)

    cate_col = x[:, 0:1]
    oh = (cate_col == lax.broadcasted_iota(jnp.int32, (1, 4), 1).astype(f32)
          ).astype(f32)
    cate_e = _mm(oh, cate_emb_ref[...])
    poi_e = half(poi_ref[...], par[:, 0:1])

    def softmax50(fv5):
        lg = -jnp.abs(_P * (_mm(fv5, rep) - percs))
        e = jnp.exp(lg)
        den = _mm(e, s50)
        return e * _mm(1.0 / den, rep)

    m_obs = softmax50(_mm(x, e_obs))
    m_alt = softmax50(_mm(x, e_alt))

    g_cate = _mm_t(perc_emb_ref[...], cate_emb_ref[...])
    gc_sel = _mm_t(oh, g_cate)
    gp = _mm_t(poi_e, perc_emb_ref[...])
    t_obs_c = _mm(m_obs * gc_sel, s50)
    t_obs_p = _mm(m_obs * gp, s50)
    t_alt_c = _mm(m_alt * gc_sel, s50)
    t_alt_p = _mm(m_alt * gp, s50)

    dots = []
    for j in range(6):
        c = half(cbg_ref[j], par[:, j + 1:j + 2])
        sgn = 1.0 if j == 0 else -1.0
        dots.append(_mm(cate_e * c, ones_d) * sgn)
        dots.append(_mm(poi_e * c, ones_d) * sgn)
    packed = jnp.concatenate(
        dots + [t_obs_c, t_obs_p, -t_alt_c, -t_alt_p], axis=1)

    col = lax.broadcasted_iota(jnp.int32, (1, 32), 1)
    w = jnp.where((col >= 2) & (col < 12), 0.2, 1.0)
    total = -jnp.sum(w * _log_sigmoid(packed))

    @pl.when(pl.program_id(0) == 0)
    def _init():
        out_ref[0, 0] = 0.0

    out_ref[0, 0] += total


_loss_tc = pl.pallas_call(
    _loss_body,
    grid=(_GRID,),
    in_specs=[
        pl.BlockSpec((_BLK, 18), lambda i: (i, 0)),
        pl.BlockSpec((4, _D), lambda i: (0, 0)),
        pl.BlockSpec((5 * _P, _D), lambda i: (0, 0)),
        pl.BlockSpec((_BLK, 2 * _D), lambda i: (i, 0)),
        pl.BlockSpec((6, _BLK, 2 * _D), lambda i: (0, i, 0)),
        pl.BlockSpec((_BLK, 8), lambda i: (i, 0)),
    ],
    out_specs=pl.BlockSpec(
        (1, 1), lambda i: (0, 0), memory_space=pltpu.SMEM),
    out_shape=jax.ShapeDtypeStruct((1, 1), jnp.float32),
    compiler_params=pltpu.CompilerParams(
        dimension_semantics=("arbitrary",)),
)


def kernel(inputs, cate_emb, poi_emb, cbg_emb, perc_emb):
    poi_ids = inputs[:, 1].astype(jnp.int32)
    cbg_cols = [2, 8, 10, 12, 14, 16]
    cbg_ids = jnp.concatenate(
        [inputs[:, c] for c in cbg_cols]).astype(jnp.int32)

    def rowof(i):
        return (i >> 12) * _PR + (i & (_PR - 1))

    def parof(i):
        return ((i >> 11) & 1).astype(jnp.float32)

    poi_pair = _make_pair(_POI_GRID)(poi_emb.T)
    cbg_pair = _make_pair(_CBG_GRID)(cbg_emb.T)
    poi_rows, cbg_rows = _gather_sc()(
        poi_pair, cbg_pair, rowof(poi_ids), rowof(cbg_ids))
    cbg_rows = cbg_rows.reshape(6, _B, 2 * _D)
    par = jnp.stack(
        [parof(poi_ids)] + [parof(cbg_ids[j * _B:(j + 1) * _B])
                            for j in range(6)]
        + [jnp.zeros((_B,), jnp.float32)], axis=1)
    out = _loss_tc(inputs, cate_emb, perc_emb, poi_rows, cbg_rows, par)
    return out[0, 0]


# speedup vs baseline: 1.9526x; 1.1528x over previous
import functools

import jax
import jax.numpy as jnp
from jax import lax
from jax.experimental import pallas as pl
from jax.experimental.pallas import tpu as pltpu
from jax.experimental.pallas import tpu_sc as plsc

_B = 16384
_P = 10
_D = 64
_NW = 32

_POI_PER_W = _B // _NW
_CBG_TOT = 6 * _B
_CBG_PER_W = _CBG_TOT // _NW
_CHUNK = 1024

_BLK = 2048
_GRID = _B // _BLK


_W = 512
_PC = 8192
_PR = _PC // 2
_PCS = _PC.bit_length() - 1

_CBG_GRID = (1000000 + _PC - 1) // _PC
_POI_GRID = (100000 + _PC - 1) // _PC


def _pair_body(tin_ref, out_ref):
    x = tin_ref[...]
    eye = (lax.broadcasted_iota(jnp.int32, (_D, _D), 0)
           == lax.broadcasted_iota(jnp.int32, (_D, _D), 1)).astype(jnp.float32)
    xt = jax.lax.dot_general(x, eye, (((0,), (0,)), ((), ())),
                             preferred_element_type=jnp.float32)
    out_ref[...] = jnp.concatenate([xt[:_PR], xt[_PR:]], axis=1)


def _make_pair(grid):
    return pl.pallas_call(
        _pair_body,
        grid=(grid,),
        in_specs=[pl.BlockSpec((_D, _PC), lambda i: (0, i))],
        out_specs=pl.BlockSpec((_PR, 2 * _D), lambda i: (i, 0)),
        out_shape=jax.ShapeDtypeStruct((grid * _PR, 2 * _D), jnp.float32),
        compiler_params=pltpu.CompilerParams(
            dimension_semantics=("parallel",)),
    )


def _sg_block(table, idx_hbm, out_hbm, base, idxv, rows, sem):
    pltpu.sync_copy(idx_hbm.at[pl.ds(base, _W)], idxv)
    pltpu.async_copy(table.at[idxv], rows, sem).wait()
    pltpu.sync_copy(rows, out_hbm.at[pl.ds(base, _W)])


def _gather_body(poi_pair, cbg_pair, poi_idx, cbg_idx, poi_out, cbg_out,
                 idxv, rows, sem):
    wid = lax.axis_index("s") * 2 + lax.axis_index("c")
    _sg_block(poi_pair, poi_idx, poi_out, wid * _W, idxv, rows, sem)
    for j in range(6):
        _sg_block(cbg_pair, cbg_idx, cbg_out, j * _B + wid * _W,
                  idxv, rows, sem)


@functools.cache
def _gather_sc():
    return pl.kernel(
        _gather_body,
        mesh=plsc.VectorSubcoreMesh(core_axis_name="c", subcore_axis_name="s"),
        out_type=[
            jax.ShapeDtypeStruct((_B, 2 * _D), jnp.float32),
            jax.ShapeDtypeStruct((_CBG_TOT, 2 * _D), jnp.float32),
        ],
        scratch_types=[
            pltpu.VMEM((_W,), jnp.int32),
            pltpu.VMEM((_W, 2 * _D), jnp.float32),
            pltpu.SemaphoreType.DMA,
        ],
        compiler_params=pltpu.CompilerParams(use_tc_tiling_on_sc=True),
    )


def _log_sigmoid(t):
    return jnp.minimum(t, 0.0) - jnp.log(1.0 + jnp.exp(-jnp.abs(t)))


def _mm(a, b):
    return jax.lax.dot_general(a, b, (((1,), (0,)), ((), ())),
                               preferred_element_type=jnp.float32)


def _mm_t(a, b):
    return jax.lax.dot_general(a, b, (((1,), (1,)), ((), ())),
                               preferred_element_type=jnp.float32)


def _loss_body(x_ref, cate_emb_ref, perc_emb_ref, poi_ref, cbg_ref, par_ref,
               out_ref):
    f32 = jnp.float32
    x = x_ref[...]
    par = par_ref[...]

    def half(xfull, p):
        return jnp.where(p > 0.5, xfull[:, _D:2 * _D], xfull[:, 0:_D])

    r18 = lax.broadcasted_iota(jnp.int32, (18, 5), 0)
    c18 = lax.broadcasted_iota(jnp.int32, (18, 5), 1)
    e_obs = (r18 == 3 + c18).astype(f32)
    e_alt = (r18 == 9 + 2 * c18).astype(f32)
    r5 = lax.broadcasted_iota(jnp.int32, (5, 5 * _P), 0)
    c5 = lax.broadcasted_iota(jnp.int32, (5, 5 * _P), 1)
    rep = (c5 // _P == r5).astype(f32)
    s50 =(lax.broadcasted_iota(jnp.int32, (5 * _P, 5), 0) // _P
           == lax.broadcasted_iota(jnp.int32, (5 * _P, 5), 1)).astype(f32)
    percs = (1.0 / (2.0 * _P)
             + (lax.broadcasted_iota(jnp.int32, (1, 5 * _P), 1) % _P
                ).astype(f32) / _P)
    ones_d = jnp.ones((_D, 1), f32)

    cate_col = x[:, 0:1]
    oh = (cate_col == lax.broadcasted_iota(jnp.int32, (1, 4), 1).astype(f32)
          ).astype(f32)
    cate_e = _mm(oh, cate_emb_ref[...])
    poi_e = half(poi_ref[...], par[:, 0:1])

    def softmax50(fv5):
        lg = -jnp.abs(_P * (_mm(fv5, rep) - percs))
        e = jnp.exp(lg)
        den = _mm(e, s50)
        return e * _mm(1.0 / den, rep)

    m_obs = softmax50(_mm(x, e_obs))
    m_alt = softmax50(_mm(x, e_alt))

    g_cate = _mm_t(perc_emb_ref[...], cate_emb_ref[...])
    gc_sel = _mm_t(oh, g_cate)
    gp = _mm_t(poi_e, perc_emb_ref[...])
    t_obs_c = _mm(m_obs * gc_sel, s50)
    t_obs_p = _mm(m_obs * gp, s50)
    t_alt_c = _mm(m_alt * gc_sel, s50)
    t_alt_p = _mm(m_alt * gp, s50)

    dots = []
    for j in range(6):
        c = half(cbg_ref[j], par[:, j + 1:j + 2])
        sgn = 1.0 if j == 0 else -1.0
        dots.append(_mm(cate_e * c, ones_d) * sgn)
        dots.append(_mm(poi_e * c, ones_d) * sgn)
    packed = jnp.concatenate(
        dots + [t_obs_c, t_obs_p, -t_alt_c, -t_alt_p], axis=1)

    col = lax.broadcasted_iota(jnp.int32, (1, 32), 1)
    w = jnp.where((col >= 2) & (col < 12), 0.2, 1.0)
    total = -jnp.sum(w * _log_sigmoid(packed))

    @pl.when(pl.program_id(0) == 0)
    def _init():
        out_ref[0, 0] = 0.0

    out_ref[0, 0] += total


_loss_tc = pl.pallas_call(
    _loss_body,
    grid=(_GRID,),
    in_specs=[
        pl.BlockSpec((_BLK, 18), lambda i: (i, 0)),
        pl.BlockSpec((4, _D), lambda i: (0, 0)),
        pl.BlockSpec((5 * _P, _D), lambda i: (0, 0)),
        pl.BlockSpec((_BLK, 2 * _D), lambda i: (i, 0)),
        pl.BlockSpec((6, _BLK, 2 * _D), lambda i: (0, i, 0)),
        pl.BlockSpec((_BLK, 8), lambda i: (i, 0)),
    ],
    out_specs=pl.BlockSpec(
        (1, 1), lambda i: (0, 0), memory_space=pltpu.SMEM),
    out_shape=jax.ShapeDtypeStruct((1, 1), jnp.float32),
    compiler_params=pltpu.CompilerParams(
        dimension_semantics=("arbitrary",)),
)


def kernel(inputs, cate_emb, poi_emb, cbg_emb, perc_emb):
    poi_ids = inputs[:, 1].astype(jnp.int32)
    cbg_cols = [2, 8, 10, 12, 14, 16]
    cbg_ids = jnp.concatenate(
        [inputs[:, c] for c in cbg_cols]).astype(jnp.int32)

    def rowof(i):
        return (i >> _PCS) * _PR + (i & (_PR - 1))

    def parof(i):
        return ((i >> (_PCS - 1)) & 1).astype(jnp.float32)

    poi_pair = _make_pair(_POI_GRID)(poi_emb.T)
    cbg_pair = _make_pair(_CBG_GRID)(cbg_emb.T)
    poi_rows, cbg_rows = _gather_sc()(
        poi_pair, cbg_pair, rowof(poi_ids), rowof(cbg_ids))
    cbg_rows = cbg_rows.reshape(6, _B, 2 * _D)
    par = jnp.stack(
        [parof(poi_ids)] + [parof(cbg_ids[j * _B:(j + 1) * _B])
                            for j in range(6)]
        + [jnp.zeros((_B,), jnp.float32)], axis=1)
    out = _loss_tc(inputs, cate_emb, perc_emb, poi_rows, cbg_rows, par)
    return out[0, 0]


# speedup vs baseline: 2.0098x; 1.0293x over previous
import functools

import jax
import jax.numpy as jnp
from jax import lax
from jax.experimental import pallas as pl
from jax.experimental.pallas import tpu as pltpu
from jax.experimental.pallas import tpu_sc as plsc

_B = 16384
_P = 10
_D = 64
_NW = 32

_POI_PER_W = _B // _NW
_CBG_TOT = 6 * _B
_CBG_PER_W = _CBG_TOT // _NW
_CHUNK = 1024

_BLK = 2048
_GRID = _B // _BLK


_W = 512
_PC = 8192
_PR = _PC // 2
_PCS = _PC.bit_length() - 1

_CBG_GRID = (1000000 + _PC - 1) // _PC
_POI_GRID = (100000 + _PC - 1) // _PC


def _pair_body(tin_ref, out_ref):
    x = tin_ref[...]
    eye = (lax.broadcasted_iota(jnp.int32, (_D, _D), 0)
           == lax.broadcasted_iota(jnp.int32, (_D, _D), 1)).astype(jnp.float32)
    xt = jax.lax.dot_general(x, eye, (((0,), (0,)), ((), ())),
                             preferred_element_type=jnp.float32)
    out_ref[...] = jnp.concatenate([xt[:_PR], xt[_PR:]], axis=1)


def _make_pair(grid):
    return pl.pallas_call(
        _pair_body,
        grid=(grid,),
        in_specs=[pl.BlockSpec((_D, _PC), lambda i: (0, i))],
        out_specs=pl.BlockSpec((_PR, 2 * _D), lambda i: (i, 0)),
        out_shape=jax.ShapeDtypeStruct((grid * _PR, 2 * _D), jnp.float32),
        compiler_params=pltpu.CompilerParams(
            dimension_semantics=("parallel",)),
    )


def _sg_block(table, idx_hbm, out_hbm, base, idxv, rows, sem):
    pltpu.sync_copy(idx_hbm.at[pl.ds(base, _W)], idxv)
    pltpu.async_copy(table.at[idxv], rows, sem).wait()
    pltpu.sync_copy(rows, out_hbm.at[pl.ds(base, _W)])


def _gather_poi_body(poi_pair, poi_idx, poi_out, idxv, rows, sem):
    wid = lax.axis_index("s") * 2 + lax.axis_index("c")
    _sg_block(poi_pair, poi_idx, poi_out, wid * _W, idxv, rows, sem)


def _gather_cbg_body(cbg_pair, cbg_idx, cbg_out, idxv, rows, sem):
    wid = lax.axis_index("s") * 2 + lax.axis_index("c")
    for j in range(6):
        _sg_block(cbg_pair, cbg_idx, cbg_out, j * _B + wid * _W,
                  idxv, rows, sem)


@functools.cache
def _gather_sc():
    scratch = [
        pltpu.VMEM((_W,), jnp.int32),
        pltpu.VMEM((_W, 2 * _D), jnp.float32),
        pltpu.SemaphoreType.DMA,
    ]
    mesh = plsc.VectorSubcoreMesh(core_axis_name="c", subcore_axis_name="s")
    params = pltpu.CompilerParams(use_tc_tiling_on_sc=True)
    poi_k = pl.kernel(
        _gather_poi_body, mesh=mesh,
        out_type=[jax.ShapeDtypeStruct((_B, 2 * _D), jnp.float32)],
        scratch_types=scratch, compiler_params=params)
    cbg_k = pl.kernel(
        _gather_cbg_body, mesh=mesh,
        out_type=[jax.ShapeDtypeStruct((_CBG_TOT, 2 * _D), jnp.float32)],
        scratch_types=scratch, compiler_params=params)
    return poi_k, cbg_k


def _log_sigmoid(t):
    return jnp.minimum(t, 0.0) - jnp.log(1.0 + jnp.exp(-jnp.abs(t)))


def _mm(a, b):
    return jax.lax.dot_general(a, b, (((1,), (0,)), ((), ())),
                               preferred_element_type=jnp.float32)


def _mm_t(a, b):
    return jax.lax.dot_general(a, b, (((1,), (1,)), ((), ())),
                               preferred_element_type=jnp.float32)


def _loss_body(x_ref, cate_emb_ref, perc_emb_ref, poi_ref, cbg_ref, par_ref,
               out_ref):
    f32 = jnp.float32
    x = x_ref[...]
    par = par_ref[...]

    def half(xfull, p):
        return jnp.where(p > 0.5, xfull[:, _D:2 * _D], xfull[:, 0:_D])

    r18 = lax.broadcasted_iota(jnp.int32, (18, 5), 0)
    c18 = lax.broadcasted_iota(jnp.int32, (18, 5), 1)
    e_obs = (r18 == 3 + c18).astype(f32)
    e_alt = (r18 == 9 + 2 * c18).astype(f32)
    r5 = lax.broadcasted_iota(jnp.int32, (5, 5 * _P), 0)
    c5 = lax.broadcasted_iota(jnp.int32, (5, 5 * _P), 1)
    rep = (c5 // _P == r5).astype(f32)
    s50 =(lax.broadcasted_iota(jnp.int32, (5 * _P, 5), 0) // _P
           == lax.broadcasted_iota(jnp.int32, (5 * _P, 5), 1)).astype(f32)
    percs = (1.0 / (2.0 * _P)
             + (lax.broadcasted_iota(jnp.int32, (1, 5 * _P), 1) % _P
                ).astype(f32) / _P)
    ones_d = jnp.ones((_D, 1), f32)

    cate_col = x[:, 0:1]
    oh = (cate_col == lax.broadcasted_iota(jnp.int32, (1, 4), 1).astype(f32)
          ).astype(f32)
    cate_e = _mm(oh, cate_emb_ref[...])
    poi_e = half(poi_ref[...], par[:, 0:1])

    def softmax50(fv5):
        lg = -jnp.abs(_P * (_mm(fv5, rep) - percs))
        e = jnp.exp(lg)
        den = _mm(e, s50)
        return e * _mm(1.0 / den, rep)

    m_obs = softmax50(_mm(x, e_obs))
    m_alt = softmax50(_mm(x, e_alt))

    g_cate = _mm_t(perc_emb_ref[...], cate_emb_ref[...])
    gc_sel = _mm_t(oh, g_cate)
    gp = _mm_t(poi_e, perc_emb_ref[...])
    t_obs_c = _mm(m_obs * gc_sel, s50)
    t_obs_p = _mm(m_obs * gp, s50)
    t_alt_c = _mm(m_alt * gc_sel, s50)
    t_alt_p = _mm(m_alt * gp, s50)

    dots = []
    for j in range(6):
        c = half(cbg_ref[j], par[:, j + 1:j + 2])
        sgn = 1.0 if j == 0 else -1.0
        dots.append(_mm(cate_e * c, ones_d) * sgn)
        dots.append(_mm(poi_e * c, ones_d) * sgn)
    packed = jnp.concatenate(
        dots + [t_obs_c, t_obs_p, -t_alt_c, -t_alt_p], axis=1)

    col = lax.broadcasted_iota(jnp.int32, (1, 32), 1)
    w = jnp.where((col >= 2) & (col < 12), 0.2, 1.0)
    total = -jnp.sum(w * _log_sigmoid(packed))

    @pl.when(pl.program_id(0) == 0)
    def _init():
        out_ref[0, 0] = 0.0

    out_ref[0, 0] += total


_loss_tc = pl.pallas_call(
    _loss_body,
    grid=(_GRID,),
    in_specs=[
        pl.BlockSpec((_BLK, 18), lambda i: (i, 0)),
        pl.BlockSpec((4, _D), lambda i: (0, 0)),
        pl.BlockSpec((5 * _P, _D), lambda i: (0, 0)),
        pl.BlockSpec((_BLK, 2 * _D), lambda i: (i, 0)),
        pl.BlockSpec((6, _BLK, 2 * _D), lambda i: (0, i, 0)),
        pl.BlockSpec((_BLK, 8), lambda i: (i, 0)),
    ],
    out_specs=pl.BlockSpec(
        (1, 1), lambda i: (0, 0), memory_space=pltpu.SMEM),
    out_shape=jax.ShapeDtypeStruct((1, 1), jnp.float32),
    compiler_params=pltpu.CompilerParams(
        dimension_semantics=("arbitrary",)),
)


def kernel(inputs, cate_emb, poi_emb, cbg_emb, perc_emb):
    poi_ids = inputs[:, 1].astype(jnp.int32)
    cbg_cols = [2, 8, 10, 12, 14, 16]
    cbg_ids = jnp.concatenate(
        [inputs[:, c] for c in cbg_cols]).astype(jnp.int32)

    def rowof(i):
        return (i >> _PCS) * _PR + (i & (_PR - 1))

    def parof(i):
        return ((i >> (_PCS - 1)) & 1).astype(jnp.float32)

    poi_k, cbg_k = _gather_sc()
    cbg_pair = _make_pair(_CBG_GRID)(cbg_emb.T)
    (cbg_rows,) = cbg_k(cbg_pair, rowof(cbg_ids))
    poi_pair = _make_pair(_POI_GRID)(poi_emb.T)
    (poi_rows,) = poi_k(poi_pair, rowof(poi_ids))
    cbg_rows = cbg_rows.reshape(6, _B, 2 * _D)
    par = jnp.stack(
        [parof(poi_ids)] + [parof(cbg_ids[j * _B:(j + 1) * _B])
                            for j in range(6)]
        + [jnp.zeros((_B,), jnp.float32)], axis=1)
    out = _loss_tc(inputs, cate_emb, perc_emb, poi_rows, cbg_rows, par)
    return out[0, 0]


# speedup vs baseline: 2.1675x; 1.0785x over previous
import functools

import jax
import jax.numpy as jnp
from jax import lax
from jax.experimental import pallas as pl
from jax.experimental.pallas import tpu as pltpu
from jax.experimental.pallas import tpu_sc as plsc

_B = 16384
_P = 10
_D = 64
_NW = 32

_POI_PER_W = _B // _NW
_CBG_TOT = 6 * _B
_CBG_PER_W = _CBG_TOT // _NW
_CHUNK = 1024

_BLK = 2048
_GRID = _B // _BLK


_W = 512
_PC = 16384
_PR = _PC // 2
_PCS = _PC.bit_length() - 1

_CBG_GRID = (1000000 + _PC - 1) // _PC
_POI_GRID = (100000 + _PC - 1) // _PC


def _pair_body(tin_ref, out_ref):
    x = tin_ref[...]
    eye = (lax.broadcasted_iota(jnp.int32, (_D, _D), 0)
           == lax.broadcasted_iota(jnp.int32, (_D, _D), 1)).astype(jnp.float32)
    xt = jax.lax.dot_general(x, eye, (((0,), (0,)), ((), ())),
                             preferred_element_type=jnp.float32)
    out_ref[...] = jnp.concatenate([xt[:_PR], xt[_PR:]], axis=1)


def _make_pair(grid):
    return pl.pallas_call(
        _pair_body,
        grid=(grid,),
        in_specs=[pl.BlockSpec((_D, _PC), lambda i: (0, i))],
        out_specs=pl.BlockSpec((_PR, 2 * _D), lambda i: (i, 0)),
        out_shape=jax.ShapeDtypeStruct((grid * _PR, 2 * _D), jnp.float32),
        compiler_params=pltpu.CompilerParams(
            dimension_semantics=("parallel",)),
    )


def _sg_block(table, idx_hbm, out_hbm, base, idxv, rows, sem):
    pltpu.sync_copy(idx_hbm.at[pl.ds(base, _W)], idxv)
    pltpu.async_copy(table.at[idxv], rows, sem).wait()
    pltpu.sync_copy(rows, out_hbm.at[pl.ds(base, _W)])


def _gather_poi_body(poi_pair, poi_idx, poi_out, idxv, rows, sem):
    wid = lax.axis_index("s") * 2 + lax.axis_index("c")
    _sg_block(poi_pair, poi_idx, poi_out, wid * _W, idxv, rows, sem)


def _gather_cbg_body(cbg_pair, cbg_idx, cbg_out, idxv, rows, sem):
    wid = lax.axis_index("s") * 2 + lax.axis_index("c")
    for j in range(6):
        _sg_block(cbg_pair, cbg_idx, cbg_out, j * _B + wid * _W,
                  idxv, rows, sem)


@functools.cache
def _gather_sc():
    scratch = [
        pltpu.VMEM((_W,), jnp.int32),
        pltpu.VMEM((_W, 2 * _D), jnp.float32),
        pltpu.SemaphoreType.DMA,
    ]
    mesh = plsc.VectorSubcoreMesh(core_axis_name="c", subcore_axis_name="s")
    params = pltpu.CompilerParams(use_tc_tiling_on_sc=True)
    poi_k = pl.kernel(
        _gather_poi_body, mesh=mesh,
        out_type=[jax.ShapeDtypeStruct((_B, 2 * _D), jnp.float32)],
        scratch_types=scratch, compiler_params=params)
    cbg_k = pl.kernel(
        _gather_cbg_body, mesh=mesh,
        out_type=[jax.ShapeDtypeStruct((_CBG_TOT, 2 * _D), jnp.float32)],
        scratch_types=scratch, compiler_params=params)
    return poi_k, cbg_k


def _log_sigmoid(t):
    return jnp.minimum(t, 0.0) - jnp.log(1.0 + jnp.exp(-jnp.abs(t)))


def _mm(a, b):
    return jax.lax.dot_general(a, b, (((1,), (0,)), ((), ())),
                               preferred_element_type=jnp.float32)


def _mm_t(a, b):
    return jax.lax.dot_general(a, b, (((1,), (1,)), ((), ())),
                               preferred_element_type=jnp.float32)


def _loss_body(x_ref, cate_emb_ref, perc_emb_ref, poi_ref, cbg_ref, par_ref,
               out_ref):
    f32 = jnp.float32
    x = x_ref[...]
    par = par_ref[...]

    def half(xfull, p):
        return jnp.where(p > 0.5, xfull[:, _D:2 * _D], xfull[:, 0:_D])

    r18 = lax.broadcasted_iota(jnp.int32, (18, 5), 0)
    c18 = lax.broadcasted_iota(jnp.int32, (18, 5), 1)
    e_obs = (r18 == 3 + c18).astype(f32)
    e_alt = (r18 == 9 + 2 * c18).astype(f32)
    r5 = lax.broadcasted_iota(jnp.int32, (5, 5 * _P), 0)
    c5 = lax.broadcasted_iota(jnp.int32, (5, 5 * _P), 1)
    rep = (c5 // _P == r5).astype(f32)
    s50 =(lax.broadcasted_iota(jnp.int32, (5 * _P, 5), 0) // _P
           == lax.broadcasted_iota(jnp.int32, (5 * _P, 5), 1)).astype(f32)
    percs = (1.0 / (2.0 * _P)
             + (lax.broadcasted_iota(jnp.int32, (1, 5 * _P), 1) % _P
                ).astype(f32) / _P)
    ones_d = jnp.ones((_D, 1), f32)

    cate_col = x[:, 0:1]
    oh = (cate_col == lax.broadcasted_iota(jnp.int32, (1, 4), 1).astype(f32)
          ).astype(f32)
    cate_e = _mm(oh, cate_emb_ref[...])
    poi_e = half(poi_ref[...], par[:, 0:1])

    def softmax50(fv5):
        lg = -jnp.abs(_P * (_mm(fv5, rep) - percs))
        e = jnp.exp(lg)
        den = _mm(e, s50)
        return e * _mm(1.0 / den, rep)

    m_obs = softmax50(_mm(x, e_obs))
    m_alt = softmax50(_mm(x, e_alt))

    g_cate = _mm_t(perc_emb_ref[...], cate_emb_ref[...])
    gc_sel = _mm_t(oh, g_cate)
    gp = _mm_t(poi_e, perc_emb_ref[...])
    t_obs_c = _mm(m_obs * gc_sel, s50)
    t_obs_p = _mm(m_obs * gp, s50)
    t_alt_c = _mm(m_alt * gc_sel, s50)
    t_alt_p = _mm(m_alt * gp, s50)

    dots = []
    for j in range(6):
        c = half(cbg_ref[j], par[:, j + 1:j + 2])
        sgn = 1.0 if j == 0 else -1.0
        dots.append(_mm(cate_e * c, ones_d) * sgn)
        dots.append(_mm(poi_e * c, ones_d) * sgn)
    packed = jnp.concatenate(
        dots + [t_obs_c, t_obs_p, -t_alt_c, -t_alt_p], axis=1)

    col = lax.broadcasted_iota(jnp.int32, (1, 32), 1)
    w = jnp.where((col >= 2) & (col < 12), 0.2, 1.0)
    total = -jnp.sum(w * _log_sigmoid(packed))

    @pl.when(pl.program_id(0) == 0)
    def _init():
        out_ref[0, 0] = 0.0

    out_ref[0, 0] += total


_loss_tc = pl.pallas_call(
    _loss_body,
    grid=(_GRID,),
    in_specs=[
        pl.BlockSpec((_BLK, 18), lambda i: (i, 0)),
        pl.BlockSpec((4, _D), lambda i: (0, 0)),
        pl.BlockSpec((5 * _P, _D), lambda i: (0, 0)),
        pl.BlockSpec((_BLK, 2 * _D), lambda i: (i, 0)),
        pl.BlockSpec((6, _BLK, 2 * _D), lambda i: (0, i, 0)),
        pl.BlockSpec((_BLK, 8), lambda i: (i, 0)),
    ],
    out_specs=pl.BlockSpec(
        (1, 1), lambda i: (0, 0), memory_space=pltpu.SMEM),
    out_shape=jax.ShapeDtypeStruct((1, 1), jnp.float32),
    compiler_params=pltpu.CompilerParams(
        dimension_semantics=("arbitrary",)),
)


def kernel(inputs, cate_emb, poi_emb, cbg_emb, perc_emb):
    poi_ids = inputs[:, 1].astype(jnp.int32)
    cbg_cols = [2, 8, 10, 12, 14, 16]
    cbg_ids = jnp.concatenate(
        [inputs[:, c] for c in cbg_cols]).astype(jnp.int32)

    def rowof(i):
        return (i >> _PCS) * _PR + (i & (_PR - 1))

    def parof(i):
        return ((i >> (_PCS - 1)) & 1).astype(jnp.float32)

    poi_k, cbg_k = _gather_sc()
    cbg_pair = _make_pair(_CBG_GRID)(cbg_emb.T)
    (cbg_rows,) = cbg_k(cbg_pair, rowof(cbg_ids))
    poi_pair = _make_pair(_POI_GRID)(poi_emb.T)
    (poi_rows,) = poi_k(poi_pair, rowof(poi_ids))
    cbg_rows = cbg_rows.reshape(6, _B, 2 * _D)
    par = jnp.stack(
        [parof(poi_ids)] + [parof(cbg_ids[j * _B:(j + 1) * _B])
                            for j in range(6)]
        + [jnp.zeros((_B,), jnp.float32)], axis=1)
    out = _loss_tc(inputs, cate_emb, perc_emb, poi_rows, cbg_rows, par)
    return out[0, 0]


# speedup vs baseline: 2.2415x; 1.0341x over previous
import functools

import jax
import jax.numpy as jnp
from jax import lax
from jax.experimental import pallas as pl
from jax.experimental.pallas import tpu as pltpu
from jax.experimental.pallas import tpu_sc as plsc

_B = 16384
_P = 10
_D = 64
_NW = 32

_POI_PER_W = _B // _NW
_CBG_TOT = 6 * _B
_CBG_PER_W = _CBG_TOT // _NW
_CHUNK = 1024

_BLK = 2048
_GRID = _B // _BLK


_W = 512
_PC = 32768
_PR = _PC // 2
_PCS = _PC.bit_length() - 1

_CBG_GRID = (1000000 + _PC - 1) // _PC
_POI_GRID = (100000 + _PC - 1) // _PC


def _pair_body(tin_ref, out_ref):
    x = tin_ref[...]
    eye = (lax.broadcasted_iota(jnp.int32, (_D, _D), 0)
           == lax.broadcasted_iota(jnp.int32, (_D, _D), 1)).astype(jnp.float32)
    xt = jax.lax.dot_general(x, eye, (((0,), (0,)), ((), ())),
                             preferred_element_type=jnp.float32)
    out_ref[...] = jnp.concatenate([xt[:_PR], xt[_PR:]], axis=1)


def _make_pair(grid):
    return pl.pallas_call(
        _pair_body,
        grid=(grid,),
        in_specs=[pl.BlockSpec((_D, _PC), lambda i: (0, i))],
        out_specs=pl.BlockSpec((_PR, 2 * _D), lambda i: (i, 0)),
        out_shape=jax.ShapeDtypeStruct((grid * _PR, 2 * _D), jnp.float32),
        compiler_params=pltpu.CompilerParams(
            dimension_semantics=("parallel",)),
    )


def _sg_block(table, idx_hbm, out_hbm, base, idxv, rows, sem):
    pltpu.sync_copy(idx_hbm.at[pl.ds(base, _W)], idxv)
    pltpu.async_copy(table.at[idxv], rows, sem).wait()
    pltpu.sync_copy(rows, out_hbm.at[pl.ds(base, _W)])


def _gather_poi_body(poi_pair, poi_idx, poi_out, idxv, rows, sem):
    wid = lax.axis_index("s") * 2 + lax.axis_index("c")
    _sg_block(poi_pair, poi_idx, poi_out, wid * _W, idxv, rows, sem)


def _gather_cbg_body(cbg_pair, cbg_idx, cbg_out, idxv, rows, sem):
    wid = lax.axis_index("s") * 2 + lax.axis_index("c")
    for j in range(6):
        _sg_block(cbg_pair, cbg_idx, cbg_out, j * _B + wid * _W,
                  idxv, rows, sem)


@functools.cache
def _gather_sc():
    scratch = [
        pltpu.VMEM((_W,), jnp.int32),
        pltpu.VMEM((_W, 2 * _D), jnp.float32),
        pltpu.SemaphoreType.DMA,
    ]
    mesh = plsc.VectorSubcoreMesh(core_axis_name="c", subcore_axis_name="s")
    params = pltpu.CompilerParams(use_tc_tiling_on_sc=True)
    poi_k = pl.kernel(
        _gather_poi_body, mesh=mesh,
        out_type=[jax.ShapeDtypeStruct((_B, 2 * _D), jnp.float32)],
        scratch_types=scratch, compiler_params=params)
    cbg_k = pl.kernel(
        _gather_cbg_body, mesh=mesh,
        out_type=[jax.ShapeDtypeStruct((_CBG_TOT, 2 * _D), jnp.float32)],
        scratch_types=scratch, compiler_params=params)
    return poi_k, cbg_k


def _log_sigmoid(t):
    return jnp.minimum(t, 0.0) - jnp.log(1.0 + jnp.exp(-jnp.abs(t)))


def _mm(a, b):
    return jax.lax.dot_general(a, b, (((1,), (0,)), ((), ())),
                               preferred_element_type=jnp.float32)


def _mm_t(a, b):
    return jax.lax.dot_general(a, b, (((1,), (1,)), ((), ())),
                               preferred_element_type=jnp.float32)


def _loss_body(x_ref, cate_emb_ref, perc_emb_ref, poi_ref, cbg_ref, par_ref,
               out_ref):
    f32 = jnp.float32
    x = x_ref[...]
    par = par_ref[...]

    def half(xfull, p):
        return jnp.where(p > 0.5, xfull[:, _D:2 * _D], xfull[:, 0:_D])

    r18 = lax.broadcasted_iota(jnp.int32, (18, 5), 0)
    c18 = lax.broadcasted_iota(jnp.int32, (18, 5), 1)
    e_obs = (r18 == 3 + c18).astype(f32)
    e_alt = (r18 == 9 + 2 * c18).astype(f32)
    r5 = lax.broadcasted_iota(jnp.int32, (5, 5 * _P), 0)
    c5 = lax.broadcasted_iota(jnp.int32, (5, 5 * _P), 1)
    rep = (c5 // _P == r5).astype(f32)
    s50 =(lax.broadcasted_iota(jnp.int32, (5 * _P, 5), 0) // _P
           == lax.broadcasted_iota(jnp.int32, (5 * _P, 5), 1)).astype(f32)
    percs = (1.0 / (2.0 * _P)
             + (lax.broadcasted_iota(jnp.int32, (1, 5 * _P), 1) % _P
                ).astype(f32) / _P)
    ones_d = jnp.ones((_D, 1), f32)

    cate_col = x[:, 0:1]
    oh = (cate_col == lax.broadcasted_iota(jnp.int32, (1, 4), 1).astype(f32)
          ).astype(f32)
    cate_e = _mm(oh, cate_emb_ref[...])
    poi_e = half(poi_ref[...], par[:, 0:1])

    def softmax50(fv5):
        lg = -jnp.abs(_P * (_mm(fv5, rep) - percs))
        e = jnp.exp(lg)
        den = _mm(e, s50)
        return e * _mm(1.0 / den, rep)

    m_obs = softmax50(_mm(x, e_obs))
    m_alt = softmax50(_mm(x, e_alt))

    g_cate = _mm_t(perc_emb_ref[...], cate_emb_ref[...])
    gc_sel = _mm_t(oh, g_cate)
    gp = _mm_t(poi_e, perc_emb_ref[...])
    t_obs_c = _mm(m_obs * gc_sel, s50)
    t_obs_p = _mm(m_obs * gp, s50)
    t_alt_c = _mm(m_alt * gc_sel, s50)
    t_alt_p = _mm(m_alt * gp, s50)

    dots = []
    for j in range(6):
        c = half(cbg_ref[j], par[:, j + 1:j + 2])
        sgn = 1.0 if j == 0 else -1.0
        dots.append(_mm(cate_e * c, ones_d) * sgn)
        dots.append(_mm(poi_e * c, ones_d) * sgn)
    packed = jnp.concatenate(
        dots + [t_obs_c, t_obs_p, -t_alt_c, -t_alt_p], axis=1)

    col = lax.broadcasted_iota(jnp.int32, (1, 32), 1)
    w = jnp.where((col >= 2) & (col < 12), 0.2, 1.0)
    total = -jnp.sum(w * _log_sigmoid(packed))

    @pl.when(pl.program_id(0) == 0)
    def _init():
        out_ref[0, 0] = 0.0

    out_ref[0, 0] += total


_loss_tc = pl.pallas_call(
    _loss_body,
    grid=(_GRID,),
    in_specs=[
        pl.BlockSpec((_BLK, 18), lambda i: (i, 0)),
        pl.BlockSpec((4, _D), lambda i: (0, 0)),
        pl.BlockSpec((5 * _P, _D), lambda i: (0, 0)),
        pl.BlockSpec((_BLK, 2 * _D), lambda i: (i, 0)),
        pl.BlockSpec((6, _BLK, 2 * _D), lambda i: (0, i, 0)),
        pl.BlockSpec((_BLK, 8), lambda i: (i, 0)),
    ],
    out_specs=pl.BlockSpec(
        (1, 1), lambda i: (0, 0), memory_space=pltpu.SMEM),
    out_shape=jax.ShapeDtypeStruct((1, 1), jnp.float32),
    compiler_params=pltpu.CompilerParams(
        dimension_semantics=("arbitrary",)),
)


def kernel(inputs, cate_emb, poi_emb, cbg_emb, perc_emb):
    poi_ids = inputs[:, 1].astype(jnp.int32)
    cbg_cols = [2, 8, 10, 12, 14, 16]
    cbg_ids = jnp.concatenate(
        [inputs[:, c] for c in cbg_cols]).astype(jnp.int32)

    def rowof(i):
        return (i >> _PCS) * _PR + (i & (_PR - 1))

    def parof(i):
        return ((i >> (_PCS - 1)) & 1).astype(jnp.float32)

    poi_k, cbg_k = _gather_sc()
    cbg_pair = _make_pair(_CBG_GRID)(cbg_emb.T)
    (cbg_rows,) = cbg_k(cbg_pair, rowof(cbg_ids))
    poi_pair = _make_pair(_POI_GRID)(poi_emb.T)
    (poi_rows,) = poi_k(poi_pair, rowof(poi_ids))
    cbg_rows = cbg_rows.reshape(6, _B, 2 * _D)
    par = jnp.stack(
        [parof(poi_ids)] + [parof(cbg_ids[j * _B:(j + 1) * _B])
                            for j in range(6)]
        + [jnp.zeros((_B,), jnp.float32)], axis=1)
    out = _loss_tc(inputs, cate_emb, perc_emb, poi_rows, cbg_rows, par)
    return out[0, 0]
